# baseline (speedup 1.0000x reference)
import jax
import jax.numpy as jnp
from jax import lax
from jax.experimental import pallas as pl
from jax.experimental.pallas import tpu as pltpu
from jax.experimental.pallas import tpu_sc as plsc

MAX_POS = 8192
D_MODEL = 1024
BATCH = 4
SEQ = 8192

NC = 2
NS = 16
NW = NC * NS

B_TOTAL = BATCH * SEQ
ROWS_PER_W = B_TOTAL // NW
CHUNK = 8
N_CHUNKS = ROWS_PER_W // CHUNK
NBUF = 8
DEPTH = 4


def _gather_body(table_hbm, ids_hbm, out_hbm, idx_v, bufs, sgs, sos):
    wid = lax.axis_index("s") * NC + lax.axis_index("c")
    pltpu.sync_copy(ids_hbm.at[wid], idx_v)
    row_base = wid * ROWS_PER_W

    def gather(c, p):
        return pltpu.make_async_copy(table_hbm.at[idx_v.at[c]], bufs[p], sgs[p])

    def put(c, p):
        return pltpu.make_async_copy(
            bufs[p], out_hbm.at[pl.ds(row_base + c * CHUNK, CHUNK)], sos[p]
        )

    def step(c, p, first, last):
        gather(c, p).wait()
        put(c, p).start()
        if not first:
            put(c - DEPTH, (p + DEPTH) % NBUF).wait()
        if not last:
            gather(c + DEPTH, (p + DEPTH) % NBUF).start()

    for p in range(DEPTH):
        gather(p, p).start()
    for c in range(DEPTH):
        step(c, c, first=True, last=False)

    def group(m, carry):
        c = NBUF * m + DEPTH
        for j in range(NBUF):
            step(c + j, (DEPTH + j) % NBUF, first=False, last=False)
        return carry

    lax.fori_loop(0, (N_CHUNKS - 2 * DEPTH) // NBUF, group, 0)
    for j in range(DEPTH):
        c = N_CHUNKS - DEPTH + j
        step(c, c % NBUF, first=False, last=True)
    for j in range(DEPTH):
        c = N_CHUNKS - DEPTH + j
        put(c, c % NBUF).wait()


@jax.jit
def _sc_gather(table, ids):
    mesh = plsc.VectorSubcoreMesh(
        core_axis_name="c", subcore_axis_name="s", num_cores=NC, num_subcores=NS
    )
    f = pl.kernel(
        _gather_body,
        out_type=jax.ShapeDtypeStruct((B_TOTAL, D_MODEL), jnp.float32),
        mesh=mesh,
        scratch_types=[
            pltpu.VMEM((N_CHUNKS, CHUNK), jnp.int32),
            [pltpu.VMEM((CHUNK, D_MODEL), jnp.float32) for _ in range(NBUF)],
            [pltpu.SemaphoreType.DMA for _ in range(NBUF)],
            [pltpu.SemaphoreType.DMA for _ in range(NBUF)],
        ],
    )
    return f(table, ids)


def kernel(position_ids, table):
    ids = position_ids.astype(jnp.int32).reshape(NW, N_CHUNKS, CHUNK)
    out = _sc_gather(table, ids)
    return out.reshape(BATCH, SEQ, D_MODEL)


# speedup vs baseline: 1.0040x; 1.0040x over previous
import jax
import jax.numpy as jnp
from jax import lax
from jax.experimental import pallas as pl
from jax.experimental.pallas import tpu as pltpu
from jax.experimental.pallas import tpu_sc as plsc

MAX_POS = 8192
D_MODEL = 1024
BATCH = 4
SEQ = 8192

NC = 2
NS = 16
NW = NC * NS

B_TOTAL = BATCH * SEQ
ROWS_PER_W = B_TOTAL // NW
CHUNK = 8
N_CHUNKS = ROWS_PER_W // CHUNK
DG = 6
DP = 2
NBUF = DG + DP


def _gather_body(table_hbm, ids_hbm, out_hbm, idx_v, bufs, sgs, sos):
    wid = lax.axis_index("s") * NC + lax.axis_index("c")
    pltpu.sync_copy(ids_hbm.at[wid], idx_v)
    row_base = wid * ROWS_PER_W

    def gather(c, p):
        return pltpu.make_async_copy(table_hbm.at[idx_v.at[c]], bufs[p], sgs[p])

    def put(c, p):
        return pltpu.make_async_copy(
            bufs[p], out_hbm.at[pl.ds(row_base + c * CHUNK, CHUNK)], sos[p]
        )

    def step(c, p, head, tail):
        gather(c, p).wait()
        put(c, p).start()
        if not head:
            put(c - DP, (p + DG) % NBUF).wait()
        if not tail:
            gather(c + DG, (p + DG) % NBUF).start()

    for p in range(DG):
        gather(p, p).start()
    for c in range(DP):
        step(c, c, head=True, tail=False)

    def group(m, carry):
        c = NBUF * m + DP
        for j in range(NBUF):
            step(c + j, (DP + j) % NBUF, head=False, tail=False)
        return carry

    lax.fori_loop(0, (N_CHUNKS - NBUF) // NBUF, group, 0)
    for j in range(DG):
        c = N_CHUNKS - DG + j
        step(c, c % NBUF, head=False, tail=True)
    for j in range(DP):
        c = N_CHUNKS - DP + j
        put(c, c % NBUF).wait()


@jax.jit
def _sc_gather(table, ids):
    mesh = plsc.VectorSubcoreMesh(
        core_axis_name="c", subcore_axis_name="s", num_cores=NC, num_subcores=NS
    )
    f = pl.kernel(
        _gather_body,
        out_type=jax.ShapeDtypeStruct((B_TOTAL, D_MODEL), jnp.float32),
        mesh=mesh,
        scratch_types=[
            pltpu.VMEM((N_CHUNKS, CHUNK), jnp.int32),
            [pltpu.VMEM((CHUNK, D_MODEL), jnp.float32) for _ in range(NBUF)],
            [pltpu.SemaphoreType.DMA for _ in range(NBUF)],
            [pltpu.SemaphoreType.DMA for _ in range(NBUF)],
        ],
    )
    return f(table, ids)


def kernel(position_ids, table):
    ids = position_ids.astype(jnp.int32).reshape(NW, N_CHUNKS, CHUNK)
    out = _sc_gather(table, ids)
    return out.reshape(BATCH, SEQ, D_MODEL)
